# baseline (speedup 1.0000x reference)
import jax
import jax.numpy as jnp
from jax import lax
from jax.experimental import pallas as pl
from jax.experimental.pallas import tpu as pltpu
from jax.experimental.pallas import tpu_sc as plsc

NUM_E = 8192
DIM = 256
BETA = 0.25
N_TOK = 16384
HALF = N_TOK // 2


BM = 256
CHW = 2816
NCH = 3
CH_BASE = (0, 2736, 5472)
BIG = 2 ** 30


def _argmin_body(z_ref, embT_ref, idx_ref, en_ref):
    @pl.when(pl.program_id(0) == 0)
    def _():
        e_all = embT_ref[...]
        en_ref[...] = jnp.sum(e_all * e_all, axis=0, keepdims=True)

    z = z_ref[...]
    a = jnp.sum(z * z, axis=1, keepdims=True)
    z2 = z * 2.0
    acc_v = jnp.full((BM, 1), jnp.inf, jnp.float32)
    acc_i = jnp.full((BM, 1), BIG, jnp.int32)
    io = lax.broadcasted_iota(jnp.int32, (BM, CHW), 1).astype(jnp.float32)
    for c in range(NCH):
        e = embT_ref[:, c * CHW:(c + 1) * CHW]
        p2 = jnp.dot(z2, e, preferred_element_type=jnp.float32)
        en = en_ref[:, c * CHW:(c + 1) * CHW]
        d = (a - p2) + en
        m = jnp.min(d, axis=1, keepdims=True)
        li_f = jnp.min(jnp.where(d == m, io, float(BIG)), axis=1, keepdims=True)
        li = li_f.astype(jnp.int32) + CH_BASE[c]
        take = (m < acc_v) | ((m == acc_v) & (li < acc_i))
        acc_i = jnp.where(take, li, acc_i)
        acc_v = jnp.where(take, m, acc_v)
        acc_v = acc_v.astype(jnp.bfloat16).astype(jnp.float32)
    idx_ref[...] = acc_i


def _compute_indices(z, embT_pad, row_off, n_rows):
    ob = row_off // BM
    return pl.pallas_call(
        _argmin_body,
        grid=(n_rows // BM,),
        in_specs=[
            pl.BlockSpec((BM, DIM), lambda i: (i + ob, 0)),
            pl.BlockSpec((DIM, NCH * CHW), lambda i: (0, 0)),
        ],
        out_specs=pl.BlockSpec((BM, 1), lambda i: (i, 0)),
        out_shape=jax.ShapeDtypeStruct((n_rows, 1), jnp.int32),
        scratch_shapes=[pltpu.VMEM((1, NCH * CHW), jnp.float32)],
        compiler_params=pltpu.CompilerParams(dimension_semantics=("arbitrary",)),
    )(z, embT_pad)


def _pad_codebook(emb):
    pad = jnp.full((80, DIM), 100.0, jnp.float32)
    pad2 = jnp.full((96, DIM), 100.0, jnp.float32)
    return jnp.concatenate(
        [emb[0:2736], pad, emb[2736:5472], pad, emb[5472:8192], pad2], axis=0).T


_NW = 32
_GCH = 128


def _make_gather_body(n_rows):
    rows_per_w = n_rows // _NW
    nch = rows_per_w // _GCH

    def body(emb_hbm, idx_hbm, out_hbm, idx_v, rows_v, sem):
        wid = lax.axis_index("s") * 2 + lax.axis_index("c")

        @pl.loop(0, nch)
        def _(cc):
            base = wid * rows_per_w + cc * _GCH
            pltpu.sync_copy(idx_hbm.at[pl.ds(base, _GCH)], idx_v)
            pltpu.async_copy(emb_hbm.at[idx_v], rows_v, sem).wait()
            pltpu.sync_copy(rows_v, out_hbm.at[pl.ds(base, _GCH)])

    return body


def _gather_rows(emb, idx_flat):
    n_rows = idx_flat.shape[0]
    k = pl.kernel(
        _make_gather_body(n_rows),
        out_type=jax.ShapeDtypeStruct((n_rows, DIM), jnp.float32),
        mesh=plsc.VectorSubcoreMesh(core_axis_name="c", subcore_axis_name="s"),
        scratch_types=[
            pltpu.VMEM((_GCH,), jnp.int32),
            pltpu.VMEM((_GCH, DIM), jnp.float32),
            pltpu.SemaphoreType.DMA,
        ],
    )
    return k(emb, idx_flat)


BM2 = 2048


def _st_body(z_ref, g_ref, _buf_ref, st_ref, ls_ref):
    z = z_ref[...]
    g = g_ref[...]
    dlt = g - z
    st_ref[...] = z + dlt
    sq = dlt * dlt
    ls_ref[0] = jnp.sum(sq, axis=0, keepdims=True).sum(axis=1, keepdims=True)


def _st_loss_half(z, g, buf, row_off):
    ob = row_off // BM2
    nblk = HALF // BM2
    return pl.pallas_call(
        _st_body,
        grid=(nblk,),
        in_specs=[
            pl.BlockSpec((BM2, DIM), lambda i: (i + ob, 0)),
            pl.BlockSpec((BM2, DIM), lambda i: (i, 0)),
            pl.BlockSpec(memory_space=pl.ANY),
        ],
        out_specs=[
            pl.BlockSpec((BM2, DIM), lambda i: (i + ob, 0)),
            pl.BlockSpec((1, 1, 1), lambda i: (i, 0, 0)),
        ],
        out_shape=[
            jax.ShapeDtypeStruct((N_TOK, DIM), jnp.float32),
            jax.ShapeDtypeStruct((nblk, 1, 1), jnp.float32),
        ],
        input_output_aliases={2: 0},
        compiler_params=pltpu.CompilerParams(dimension_semantics=("arbitrary",)),
    )(z, g, buf)


def kernel(z_e, emb):
    b, d, h, w = z_e.shape
    z = jnp.transpose(z_e, (0, 2, 3, 1)).reshape(-1, d)
    embT_pad = _pad_codebook(emb)

    idx_a = _compute_indices(z, embT_pad, 0, HALF)
    idx_b = _compute_indices(z, embT_pad, HALF, HALF)
    g_a = _gather_rows(emb, idx_a[:, 0])
    g_b = _gather_rows(emb, idx_b[:, 0])

    buf = jnp.zeros((N_TOK, DIM), jnp.float32)
    st_a, part_a = _st_loss_half(z, g_a, buf, 0)
    st, part_b = _st_loss_half(z, g_b, st_a, HALF)

    total = jnp.sum(part_a) + jnp.sum(part_b)
    m = total / (b * d * h * w)
    vq_loss = m + BETA * m
    z_q_st = jnp.transpose(st.reshape(b, h, w, d), (0, 3, 1, 2))
    indices = jnp.concatenate([idx_a[:, 0], idx_b[:, 0]]).reshape(b, h, w)
    return (z_q_st, vq_loss, indices)


# speedup vs baseline: 1.0196x; 1.0196x over previous
import jax
import jax.numpy as jnp
from jax import lax
from jax.experimental import pallas as pl
from jax.experimental.pallas import tpu as pltpu
from jax.experimental.pallas import tpu_sc as plsc

NUM_E = 8192
DIM = 256
BETA = 0.25
N_TOK = 16384
HALF = N_TOK // 2


BM = 256
CHW = 2816
NCH = 3
CH_BASE = (0, 2736, 5472)
BIG = 2 ** 30


def _argmin_body(z_ref, embT_ref, idx_ref, en_ref):
    @pl.when(pl.program_id(0) == 0)
    def _():
        e_all = embT_ref[...]
        en_ref[...] = jnp.sum(e_all * e_all, axis=0, keepdims=True)

    z = z_ref[...]
    a = jnp.sum(z * z, axis=1, keepdims=True)
    z2 = z * 2.0
    acc_v = jnp.full((BM, 1), jnp.inf, jnp.float32)
    acc_i = jnp.full((BM, 1), BIG, jnp.int32)
    io = lax.broadcasted_iota(jnp.int32, (BM, CHW), 1).astype(jnp.float32)
    for c in range(NCH):
        e = embT_ref[:, c * CHW:(c + 1) * CHW]
        p2 = jnp.dot(z2, e, preferred_element_type=jnp.float32)
        en = en_ref[:, c * CHW:(c + 1) * CHW]
        d = (a - p2) + en
        m = jnp.min(d, axis=1, keepdims=True)
        li_f = jnp.min(jnp.where(d == m, io, float(BIG)), axis=1, keepdims=True)
        li = li_f.astype(jnp.int32) + CH_BASE[c]
        take = (m < acc_v) | ((m == acc_v) & (li < acc_i))
        acc_i = jnp.where(take, li, acc_i)
        acc_v = jnp.where(take, m, acc_v)
        acc_v = acc_v.astype(jnp.bfloat16).astype(jnp.float32)
    idx_ref[...] = acc_i


def _compute_indices(z, embT_pad, row_off, n_rows):
    ob = row_off // BM
    return pl.pallas_call(
        _argmin_body,
        grid=(n_rows // BM,),
        in_specs=[
            pl.BlockSpec((BM, DIM), lambda i: (i + ob, 0)),
            pl.BlockSpec((DIM, NCH * CHW), lambda i: (0, 0)),
        ],
        out_specs=pl.BlockSpec((BM, 1), lambda i: (i, 0)),
        out_shape=jax.ShapeDtypeStruct((n_rows, 1), jnp.int32),
        scratch_shapes=[pltpu.VMEM((1, NCH * CHW), jnp.float32)],
        compiler_params=pltpu.CompilerParams(dimension_semantics=("arbitrary",)),
    )(z, embT_pad)


def _pad_codebook(emb):
    pad = jnp.full((80, DIM), 100.0, jnp.float32)
    pad2 = jnp.full((96, DIM), 100.0, jnp.float32)
    return jnp.concatenate(
        [emb[0:2736], pad, emb[2736:5472], pad, emb[5472:8192], pad2], axis=0).T


_NW = 32
_GCH = 128


def _make_gather_body(n_rows):
    rows_per_w = n_rows // _NW
    nch = rows_per_w // _GCH

    def body(emb_hbm, idx_hbm, out_hbm, idx_v, rows_v, sem):
        wid = lax.axis_index("s") * 2 + lax.axis_index("c")

        @pl.loop(0, nch)
        def _(cc):
            base = wid * rows_per_w + cc * _GCH
            pltpu.sync_copy(idx_hbm.at[pl.ds(base, _GCH)], idx_v)
            pltpu.async_copy(emb_hbm.at[idx_v], rows_v, sem).wait()
            pltpu.sync_copy(rows_v, out_hbm.at[pl.ds(base, _GCH)])

    return body


def _gather_rows(emb, idx_flat):
    n_rows = idx_flat.shape[0]
    k = pl.kernel(
        _make_gather_body(n_rows),
        out_type=jax.ShapeDtypeStruct((n_rows, DIM), jnp.float32),
        mesh=plsc.VectorSubcoreMesh(core_axis_name="c", subcore_axis_name="s"),
        scratch_types=[
            pltpu.VMEM((_GCH,), jnp.int32),
            pltpu.VMEM((_GCH, DIM), jnp.float32),
            pltpu.SemaphoreType.DMA,
        ],
    )
    return k(emb, idx_flat)


BM2 = 2048


def _st_body_a(z_ref, g_ref, st_ref, ls_ref):
    _st_common(z_ref, g_ref, st_ref, ls_ref)


def _st_body_b(z_ref, g_ref, _buf_ref, st_ref, ls_ref):
    _st_common(z_ref, g_ref, st_ref, ls_ref)


def _st_common(z_ref, g_ref, st_ref, ls_ref):
    z = z_ref[...]
    g = g_ref[...]
    dlt = g - z
    st_ref[...] = z + dlt
    sq = dlt * dlt
    ls_ref[0] = jnp.sum(sq, axis=0, keepdims=True).sum(axis=1, keepdims=True)


def _st_loss_half(z, g, buf, row_off):
    ob = row_off // BM2
    nblk = HALF // BM2
    in_specs = [
        pl.BlockSpec((BM2, DIM), lambda i: (i + ob, 0)),
        pl.BlockSpec((BM2, DIM), lambda i: (i, 0)),
    ]
    args = (z, g)
    body = _st_body_a
    aliases = {}
    if buf is not None:
        in_specs.append(pl.BlockSpec(memory_space=pl.ANY))
        args = (z, g, buf)
        body = _st_body_b
        aliases = {2: 0}
    return pl.pallas_call(
        body,
        grid=(nblk,),
        in_specs=in_specs,
        out_specs=[
            pl.BlockSpec((BM2, DIM), lambda i: (i + ob, 0)),
            pl.BlockSpec((1, 1, 1), lambda i: (i, 0, 0)),
        ],
        out_shape=[
            jax.ShapeDtypeStruct((N_TOK, DIM), jnp.float32),
            jax.ShapeDtypeStruct((nblk, 1, 1), jnp.float32),
        ],
        input_output_aliases=aliases,
        compiler_params=pltpu.CompilerParams(dimension_semantics=("arbitrary",)),
    )(*args)


def kernel(z_e, emb):
    b, d, h, w = z_e.shape
    z = jnp.transpose(z_e, (0, 2, 3, 1)).reshape(-1, d)
    embT_pad = _pad_codebook(emb)

    idx_a = _compute_indices(z, embT_pad, 0, HALF)
    idx_b = _compute_indices(z, embT_pad, HALF, HALF)
    g_a = _gather_rows(emb, idx_a[:, 0])
    g_b = _gather_rows(emb, idx_b[:, 0])

    st_a, part_a = _st_loss_half(z, g_a, None, 0)
    st, part_b = _st_loss_half(z, g_b, st_a, HALF)

    total = jnp.sum(part_a) + jnp.sum(part_b)
    m = total / (b * d * h * w)
    vq_loss = m + BETA * m
    z_q_st = jnp.transpose(st.reshape(b, h, w, d), (0, 3, 1, 2))
    indices = jnp.concatenate([idx_a[:, 0], idx_b[:, 0]]).reshape(b, h, w)
    return (z_q_st, vq_loss, indices)


# speedup vs baseline: 1.0278x; 1.0081x over previous
import jax
import jax.numpy as jnp
from jax import lax
from jax.experimental import pallas as pl
from jax.experimental.pallas import tpu as pltpu
from jax.experimental.pallas import tpu_sc as plsc

NUM_E = 8192
DIM = 256
BETA = 0.25
N_TOK = 16384
HALF = N_TOK // 2


BM = 256
CHW = 2816
NCH = 3
CH_BASE = (0, 2736, 5472)
BIG = 2 ** 30


def _argmin_body(z_ref, embT_ref, idx_ref, en_ref):
    @pl.when(pl.program_id(0) == 0)
    def _():
        e_all = embT_ref[...]
        en_ref[...] = jnp.sum(e_all * e_all, axis=0, keepdims=True)

    z = z_ref[...]
    a = jnp.sum(z * z, axis=1, keepdims=True)
    z2 = z * 2.0
    acc_v = jnp.full((BM, 1), jnp.inf, jnp.float32)
    acc_i = jnp.full((BM, 1), BIG, jnp.int32)
    io = lax.broadcasted_iota(jnp.int32, (BM, CHW), 1).astype(jnp.float32)
    for c in range(NCH):
        e = embT_ref[:, c * CHW:(c + 1) * CHW]
        p2 = jnp.dot(z2, e, preferred_element_type=jnp.float32)
        en = en_ref[:, c * CHW:(c + 1) * CHW]
        d = (a - p2) + en
        m = jnp.min(d, axis=1, keepdims=True)
        li_f = jnp.min(jnp.where(d == m, io, float(BIG)), axis=1, keepdims=True)
        li = li_f.astype(jnp.int32) + CH_BASE[c]
        take = (m < acc_v) | ((m == acc_v) & (li < acc_i))
        acc_i = jnp.where(take, li, acc_i)
        acc_v = jnp.where(take, m, acc_v)
        acc_v = acc_v.astype(jnp.bfloat16).astype(jnp.float32)
    idx_ref[...] = acc_i


def _compute_indices(z, embT_pad, row_off, n_rows):
    ob = row_off // BM
    return pl.pallas_call(
        _argmin_body,
        grid=(n_rows // BM,),
        in_specs=[
            pl.BlockSpec((BM, DIM), lambda i: (i + ob, 0)),
            pl.BlockSpec((DIM, NCH * CHW), lambda i: (0, 0)),
        ],
        out_specs=pl.BlockSpec((BM, 1), lambda i: (i, 0)),
        out_shape=jax.ShapeDtypeStruct((n_rows, 1), jnp.int32),
        scratch_shapes=[pltpu.VMEM((1, NCH * CHW), jnp.float32)],
        compiler_params=pltpu.CompilerParams(dimension_semantics=("arbitrary",)),
    )(z, embT_pad)


def _pad_codebook(emb):
    pad = jnp.full((80, DIM), 100.0, jnp.float32)
    pad2 = jnp.full((96, DIM), 100.0, jnp.float32)
    return jnp.concatenate(
        [emb[0:2736], pad, emb[2736:5472], pad, emb[5472:8192], pad2], axis=0).T


_NW = 32
_GCH = 128


def _make_gather_body(n_rows):
    rows_per_w = n_rows // _NW
    nch = rows_per_w // _GCH

    def body(emb_hbm, idx_hbm, out_hbm, idx_v, rows_v, sem):
        wid = lax.axis_index("s") * 2 + lax.axis_index("c")

        @pl.loop(0, nch)
        def _(cc):
            base = wid * rows_per_w + cc * _GCH
            pltpu.sync_copy(idx_hbm.at[pl.ds(base, _GCH)], idx_v)
            pltpu.async_copy(emb_hbm.at[idx_v], rows_v, sem).wait()
            pltpu.sync_copy(rows_v, out_hbm.at[pl.ds(base, _GCH)])

    return body


def _gather_rows(emb, idx_flat):
    n_rows = idx_flat.shape[0]
    k = pl.kernel(
        _make_gather_body(n_rows),
        out_type=jax.ShapeDtypeStruct((n_rows, DIM), jnp.float32),
        mesh=plsc.VectorSubcoreMesh(core_axis_name="c", subcore_axis_name="s"),
        scratch_types=[
            pltpu.VMEM((_GCH,), jnp.int32),
            pltpu.VMEM((_GCH, DIM), jnp.float32),
            pltpu.SemaphoreType.DMA,
        ],
    )
    return k(emb, idx_flat)


BM2 = 2048


def _st_body_a(z_ref, g_ref, st_ref, ls_ref):
    z = z_ref[...]
    g = g_ref[...]
    dlt = g - z
    st_ref[...] = z + dlt
    sq = dlt * dlt
    ls_ref[0] = jnp.sum(sq, axis=0, keepdims=True).sum(axis=1, keepdims=True)


def _st_loss(z, g):
    nblk = N_TOK // BM2
    return pl.pallas_call(
        _st_body_a,
        grid=(nblk,),
        in_specs=[
            pl.BlockSpec((BM2, DIM), lambda i: (i, 0)),
            pl.BlockSpec((BM2, DIM), lambda i: (i, 0)),
        ],
        out_specs=[
            pl.BlockSpec((BM2, DIM), lambda i: (i, 0)),
            pl.BlockSpec((1, 1, 1), lambda i: (i, 0, 0)),
        ],
        out_shape=[
            jax.ShapeDtypeStruct((N_TOK, DIM), jnp.float32),
            jax.ShapeDtypeStruct((nblk, 1, 1), jnp.float32),
        ],
        compiler_params=pltpu.CompilerParams(dimension_semantics=("arbitrary",)),
    )(z, g)


def kernel(z_e, emb):
    b, d, h, w = z_e.shape
    z = jnp.transpose(z_e, (0, 2, 3, 1)).reshape(-1, d)
    embT_pad = _pad_codebook(emb)

    idx2 = _compute_indices(z, embT_pad, 0, N_TOK)
    idx_flat = idx2[:, 0]
    g = _gather_rows(emb, idx_flat)

    st, part = _st_loss(z, g)
    total = jnp.sum(part)
    m = total / (b * d * h * w)
    vq_loss = m + BETA * m
    z_q_st = jnp.transpose(st.reshape(b, h, w, d), (0, 3, 1, 2))
    return (z_q_st, vq_loss, idx_flat.reshape(b, h, w))


# speedup vs baseline: 1.0719x; 1.0428x over previous
import jax
import jax.numpy as jnp
from jax import lax
from jax.experimental import pallas as pl
from jax.experimental.pallas import tpu as pltpu
from jax.experimental.pallas import tpu_sc as plsc

NUM_E = 8192
DIM = 256
BETA = 0.25
N_TOK = 16384
HALF = N_TOK // 2


BM = 512
CHW = 2816
NCH = 3
CH_BASE = (0, 2736, 5472)
BIG = 2 ** 30


def _argmin_body(z_ref, embT_ref, idx_ref, en_ref):
    @pl.when(pl.program_id(0) == 0)
    def _():
        e_all = embT_ref[...]
        en_ref[...] = jnp.sum(e_all * e_all, axis=0, keepdims=True)

    z = z_ref[...]
    a = jnp.sum(z * z, axis=1, keepdims=True)
    z2 = z * 2.0
    acc_v = jnp.full((BM, 1), jnp.inf, jnp.float32)
    acc_i = jnp.full((BM, 1), BIG, jnp.int32)
    io = lax.broadcasted_iota(jnp.int32, (BM, CHW), 1).astype(jnp.float32)
    for c in range(NCH):
        e = embT_ref[:, c * CHW:(c + 1) * CHW]
        p2 = jnp.dot(z2, e, preferred_element_type=jnp.float32)
        en = en_ref[:, c * CHW:(c + 1) * CHW]
        d = (a - p2) + en
        m = jnp.min(d, axis=1, keepdims=True)
        li_f = jnp.min(jnp.where(d == m, io, float(BIG)), axis=1, keepdims=True)
        li = li_f.astype(jnp.int32) + CH_BASE[c]
        take = (m < acc_v) | ((m == acc_v) & (li < acc_i))
        acc_i = jnp.where(take, li, acc_i)
        acc_v = jnp.where(take, m, acc_v)
        acc_v = acc_v.astype(jnp.bfloat16).astype(jnp.float32)
    idx_ref[...] = acc_i


def _compute_indices(z, embT_pad, row_off, n_rows):
    ob = row_off // BM
    return pl.pallas_call(
        _argmin_body,
        grid=(n_rows // BM,),
        in_specs=[
            pl.BlockSpec((BM, DIM), lambda i: (i + ob, 0)),
            pl.BlockSpec((DIM, NCH * CHW), lambda i: (0, 0)),
        ],
        out_specs=pl.BlockSpec((BM, 1), lambda i: (i, 0)),
        out_shape=jax.ShapeDtypeStruct((n_rows, 1), jnp.int32),
        scratch_shapes=[pltpu.VMEM((1, NCH * CHW), jnp.float32)],
        compiler_params=pltpu.CompilerParams(dimension_semantics=("arbitrary",)),
    )(z, embT_pad)


def _pad_codebook(emb):
    pad = jnp.full((80, DIM), 100.0, jnp.float32)
    pad2 = jnp.full((96, DIM), 100.0, jnp.float32)
    return jnp.concatenate(
        [emb[0:2736], pad, emb[2736:5472], pad, emb[5472:8192], pad2], axis=0).T


_NW = 32
_GCH = 128


def _make_gather_body(n_rows):
    rows_per_w = n_rows // _NW
    nch = rows_per_w // _GCH

    def body(emb_hbm, idx_hbm, out_hbm, idx_v, rows_v, sem):
        wid = lax.axis_index("s") * 2 + lax.axis_index("c")

        @pl.loop(0, nch)
        def _(cc):
            base = wid * rows_per_w + cc * _GCH
            pltpu.sync_copy(idx_hbm.at[pl.ds(base, _GCH)], idx_v)
            pltpu.async_copy(emb_hbm.at[idx_v], rows_v, sem).wait()
            pltpu.sync_copy(rows_v, out_hbm.at[pl.ds(base, _GCH)])

    return body


def _gather_rows(emb, idx_flat):
    n_rows = idx_flat.shape[0]
    k = pl.kernel(
        _make_gather_body(n_rows),
        out_type=jax.ShapeDtypeStruct((n_rows, DIM), jnp.float32),
        mesh=plsc.VectorSubcoreMesh(core_axis_name="c", subcore_axis_name="s"),
        scratch_types=[
            pltpu.VMEM((_GCH,), jnp.int32),
            pltpu.VMEM((_GCH, DIM), jnp.float32),
            pltpu.SemaphoreType.DMA,
        ],
    )
    return k(emb, idx_flat)


BM2 = 2048


def _st_body_a(z_ref, g_ref, st_ref, ls_ref):
    z = z_ref[...]
    g = g_ref[...]
    dlt = g - z
    st_ref[...] = z + dlt
    sq = dlt * dlt
    ls_ref[0] = jnp.sum(sq, axis=0, keepdims=True).sum(axis=1, keepdims=True)


def _st_loss(z, g):
    nblk = N_TOK // BM2
    return pl.pallas_call(
        _st_body_a,
        grid=(nblk,),
        in_specs=[
            pl.BlockSpec((BM2, DIM), lambda i: (i, 0)),
            pl.BlockSpec((BM2, DIM), lambda i: (i, 0)),
        ],
        out_specs=[
            pl.BlockSpec((BM2, DIM), lambda i: (i, 0)),
            pl.BlockSpec((1, 1, 1), lambda i: (i, 0, 0)),
        ],
        out_shape=[
            jax.ShapeDtypeStruct((N_TOK, DIM), jnp.float32),
            jax.ShapeDtypeStruct((nblk, 1, 1), jnp.float32),
        ],
        compiler_params=pltpu.CompilerParams(dimension_semantics=("arbitrary",)),
    )(z, g)


def kernel(z_e, emb):
    b, d, h, w = z_e.shape
    z = jnp.transpose(z_e, (0, 2, 3, 1)).reshape(-1, d)
    embT_pad = _pad_codebook(emb)

    idx2 = _compute_indices(z, embT_pad, 0, N_TOK)
    idx_flat = idx2[:, 0]
    g = _gather_rows(emb, idx_flat)

    st, part = _st_loss(z, g)
    total = jnp.sum(part)
    m = total / (b * d * h * w)
    vq_loss = m + BETA * m
    z_q_st = jnp.transpose(st.reshape(b, h, w, d), (0, 3, 1, 2))
    return (z_q_st, vq_loss, idx_flat.reshape(b, h, w))


# speedup vs baseline: 1.1020x; 1.0281x over previous
import jax
import jax.numpy as jnp
from jax import lax
from jax.experimental import pallas as pl
from jax.experimental.pallas import tpu as pltpu
from jax.experimental.pallas import tpu_sc as plsc

NUM_E = 8192
DIM = 256
BETA = 0.25
N_TOK = 16384
HALF = N_TOK // 2


BM = 1024
CHW = 2816
NCH = 3
CH_BASE = (0, 2736, 5472)
BIG = 2 ** 30


def _argmin_body(z_ref, embT_ref, idx_ref, en_ref):
    @pl.when(pl.program_id(0) == 0)
    def _():
        e_all = embT_ref[...]
        en_ref[...] = jnp.sum(e_all * e_all, axis=0, keepdims=True)

    z = z_ref[...]
    a = jnp.sum(z * z, axis=1, keepdims=True)
    z2 = z * 2.0
    acc_v = jnp.full((BM, 1), jnp.inf, jnp.float32)
    acc_i = jnp.full((BM, 1), BIG, jnp.int32)
    io = lax.broadcasted_iota(jnp.int32, (BM, CHW), 1).astype(jnp.float32)
    for c in range(NCH):
        e = embT_ref[:, c * CHW:(c + 1) * CHW]
        p2 = jnp.dot(z2, e, preferred_element_type=jnp.float32)
        en = en_ref[:, c * CHW:(c + 1) * CHW]
        d = (a - p2) + en
        m = jnp.min(d, axis=1, keepdims=True)
        li_f = jnp.min(jnp.where(d == m, io, float(BIG)), axis=1, keepdims=True)
        li = li_f.astype(jnp.int32) + CH_BASE[c]
        take = (m < acc_v) | ((m == acc_v) & (li < acc_i))
        acc_i = jnp.where(take, li, acc_i)
        acc_v = jnp.where(take, m, acc_v)
        acc_v = acc_v.astype(jnp.bfloat16).astype(jnp.float32)
    idx_ref[...] = acc_i


def _compute_indices(z, embT_pad, row_off, n_rows):
    ob = row_off // BM
    return pl.pallas_call(
        _argmin_body,
        grid=(n_rows // BM,),
        in_specs=[
            pl.BlockSpec((BM, DIM), lambda i: (i + ob, 0)),
            pl.BlockSpec((DIM, NCH * CHW), lambda i: (0, 0)),
        ],
        out_specs=pl.BlockSpec((BM, 1), lambda i: (i, 0)),
        out_shape=jax.ShapeDtypeStruct((n_rows, 1), jnp.int32),
        scratch_shapes=[pltpu.VMEM((1, NCH * CHW), jnp.float32)],
        compiler_params=pltpu.CompilerParams(dimension_semantics=("arbitrary",)),
    )(z, embT_pad)


def _pad_codebook(emb):
    pad = jnp.full((80, DIM), 100.0, jnp.float32)
    pad2 = jnp.full((96, DIM), 100.0, jnp.float32)
    return jnp.concatenate(
        [emb[0:2736], pad, emb[2736:5472], pad, emb[5472:8192], pad2], axis=0).T


_NW = 32
_GCH = 128


def _make_gather_body(n_rows):
    rows_per_w = n_rows // _NW
    nch = rows_per_w // _GCH

    def body(emb_hbm, idx_hbm, out_hbm, idx_v, rows_v, sem):
        wid = lax.axis_index("s") * 2 + lax.axis_index("c")

        @pl.loop(0, nch)
        def _(cc):
            base = wid * rows_per_w + cc * _GCH
            pltpu.sync_copy(idx_hbm.at[pl.ds(base, _GCH)], idx_v)
            pltpu.async_copy(emb_hbm.at[idx_v], rows_v, sem).wait()
            pltpu.sync_copy(rows_v, out_hbm.at[pl.ds(base, _GCH)])

    return body


def _gather_rows(emb, idx_flat):
    n_rows = idx_flat.shape[0]
    k = pl.kernel(
        _make_gather_body(n_rows),
        out_type=jax.ShapeDtypeStruct((n_rows, DIM), jnp.float32),
        mesh=plsc.VectorSubcoreMesh(core_axis_name="c", subcore_axis_name="s"),
        scratch_types=[
            pltpu.VMEM((_GCH,), jnp.int32),
            pltpu.VMEM((_GCH, DIM), jnp.float32),
            pltpu.SemaphoreType.DMA,
        ],
    )
    return k(emb, idx_flat)


BM2 = 2048


def _st_body_a(z_ref, g_ref, st_ref, ls_ref):
    z = z_ref[...]
    g = g_ref[...]
    dlt = g - z
    st_ref[...] = z + dlt
    sq = dlt * dlt
    ls_ref[0] = jnp.sum(sq, axis=0, keepdims=True).sum(axis=1, keepdims=True)


def _st_loss(z, g):
    nblk = N_TOK // BM2
    return pl.pallas_call(
        _st_body_a,
        grid=(nblk,),
        in_specs=[
            pl.BlockSpec((BM2, DIM), lambda i: (i, 0)),
            pl.BlockSpec((BM2, DIM), lambda i: (i, 0)),
        ],
        out_specs=[
            pl.BlockSpec((BM2, DIM), lambda i: (i, 0)),
            pl.BlockSpec((1, 1, 1), lambda i: (i, 0, 0)),
        ],
        out_shape=[
            jax.ShapeDtypeStruct((N_TOK, DIM), jnp.float32),
            jax.ShapeDtypeStruct((nblk, 1, 1), jnp.float32),
        ],
        compiler_params=pltpu.CompilerParams(dimension_semantics=("arbitrary",)),
    )(z, g)


def kernel(z_e, emb):
    b, d, h, w = z_e.shape
    z = jnp.transpose(z_e, (0, 2, 3, 1)).reshape(-1, d)
    embT_pad = _pad_codebook(emb)

    idx2 = _compute_indices(z, embT_pad, 0, N_TOK)
    idx_flat = idx2[:, 0]
    g = _gather_rows(emb, idx_flat)

    st, part = _st_loss(z, g)
    total = jnp.sum(part)
    m = total / (b * d * h * w)
    vq_loss = m + BETA * m
    z_q_st = jnp.transpose(st.reshape(b, h, w, d), (0, 3, 1, 2))
    return (z_q_st, vq_loss, idx_flat.reshape(b, h, w))
